# baseline (speedup 1.0000x reference)
import functools

import jax
import jax.numpy as jnp
from jax import lax
from jax.experimental import pallas as pl
from jax.experimental.pallas import tpu as pltpu
from jax.experimental.pallas import tpu_sc as plsc

_B = 8
_N0 = 4096


def _ceil_to(x, m):
    return (x + m - 1) // m * m


_FPS_SIZES = (1024, 256, 64, 16)


def _fps_kernel(pts_ref, o1, o2, o3, o4):
    def stage(src_ref, npoint, out_ref):
        x = src_ref[:, 0, :]
        y = src_ref[:, 1, :]
        z = src_ref[:, 2, :]
        n = x.shape[1]
        iota = lax.broadcasted_iota(jnp.int32, (_B, n), 1)
        iota3 = lax.broadcasted_iota(jnp.int32, (_B, 3, npoint), 2)

        def body(i, carry):
            dist, far, acc = carry
            oh = iota == far
            cx = jnp.sum(jnp.where(oh, x, 0.0), axis=1, keepdims=True)
            cy = jnp.sum(jnp.where(oh, y, 0.0), axis=1, keepdims=True)
            cz = jnp.sum(jnp.where(oh, z, 0.0), axis=1, keepdims=True)
            c3 = jnp.concatenate([cx, cy, cz], axis=1)[:, :, None]
            acc = jnp.where(iota3 == i, c3, acc)
            d = (x - cx) ** 2 + (y - cy) ** 2 + (z - cz) ** 2
            dist = jnp.minimum(dist, d)
            m = jnp.max(dist, axis=1, keepdims=True)
            far = jnp.min(jnp.where(dist == m, iota, n), axis=1, keepdims=True)
            return dist, far, acc

        _, _, acc = lax.fori_loop(
            0, npoint, body,
            (jnp.full((_B, n), 1e10, jnp.float32), jnp.zeros((_B, 1), jnp.int32),
             jnp.zeros((_B, 3, npoint), jnp.float32)),
        )
        out_ref[...] = acc

    stage(pts_ref, _FPS_SIZES[0], o1)
    stage(o1, _FPS_SIZES[1], o2)
    stage(o2, _FPS_SIZES[2], o3)
    stage(o3, _FPS_SIZES[3], o4)


def _run_fps(points):
    return pl.pallas_call(
        _fps_kernel,
        out_shape=[jax.ShapeDtypeStruct((_B, 3, s), jnp.float32) for s in _FPS_SIZES],
    )(points)


def _ballq_kernel(r2, ns, xyz_ref, nxyz_ref, idx_ref):
    s = nxyz_ref.shape[1]
    n = xyz_ref.shape[2]
    sqr = jnp.zeros((s, n), jnp.float32)
    for c in range(3):
        a = nxyz_ref[0, :, c:c + 1]
        b = xyz_ref[0, c:c + 1, :]
        sqr = sqr + (a - b) ** 2
    iota = lax.broadcasted_iota(jnp.int32, (s, n), 1)
    vals = jnp.where(sqr <= r2, iota, n)
    cols = []
    for _ in range(ns):
        m = jnp.min(vals, axis=1, keepdims=True)
        cols.append(m)
        vals = jnp.where(vals == m, n, vals)
    idx = jnp.concatenate(cols, axis=1)
    idx = jnp.where(idx == n, jnp.broadcast_to(idx[:, 0:1], idx.shape), idx)
    idx_ref[0] = idx


def _run_ballq(radius, ns, xyz_t, nxyz_b):
    b, _, n = xyz_t.shape
    s = nxyz_b.shape[1]
    return pl.pallas_call(
        functools.partial(_ballq_kernel, radius * radius, ns),
        grid=(b,),
        in_specs=[
            pl.BlockSpec((1, 3, n), lambda i: (i, 0, 0)),
            pl.BlockSpec((1, s, 3), lambda i: (i, 0, 0)),
        ],
        out_specs=pl.BlockSpec((1, s, ns), lambda i: (i, 0, 0)),
        out_shape=jax.ShapeDtypeStruct((b, s, ns), jnp.int32),
    )(xyz_t, nxyz_b)


def _gather_rows(table, flat_idx):
    r = flat_idx.shape[0]
    cp = table.shape[1]
    info = plsc.get_sparse_core_info()
    nw = info.num_cores * info.num_subcores
    nc = info.num_cores
    per_w = r // nw
    chunk = min(per_w, 128)
    steps = per_w // chunk
    mesh = plsc.VectorSubcoreMesh(core_axis_name="c", subcore_axis_name="s")

    @functools.partial(
        pl.kernel, mesh=mesh,
        out_type=jax.ShapeDtypeStruct((r, cp), jnp.float32),
        scratch_types=[
            pltpu.VMEM((chunk,), jnp.int32),
            pltpu.VMEM((chunk, cp), jnp.float32),
            pltpu.SemaphoreType.DMA,
        ],
    )
    def gk(table_hbm, idx_hbm, out_hbm, idx_v, rows_v, sem):
        wid = lax.axis_index("s") * nc + lax.axis_index("c")
        base = wid * per_w

        def body(t, carry):
            off = base + t * chunk
            pltpu.sync_copy(idx_hbm.at[pl.ds(off, chunk)], idx_v)
            pltpu.async_copy(table_hbm.at[idx_v], rows_v, sem).wait()
            pltpu.sync_copy(rows_v, out_hbm.at[pl.ds(off, chunk)])
            return carry

        lax.fori_loop(0, steps, body, 0)

    return gk(table, flat_idx)


def _sa_mlp_kernel(ns, g, nw, rows_ref, cent_ref, *refs):
    w_refs = refs[:-1]
    out_ref = refs[-1]
    cp = rows_ref.shape[1]
    x = rows_ref[...].reshape(g, ns, cp) - cent_ref[...][:, None, :]
    x = x.reshape(g * ns, cp)
    for li in range(nw):
        w = w_refs[2 * li][...]
        bb = w_refs[2 * li + 1][...]
        x = jnp.maximum(jnp.dot(x, w, preferred_element_type=jnp.float32) + bb, 0.0)
    h = x.shape[1]
    out_ref[...] = jnp.max(x.reshape(g, ns, h), axis=1)


def _run_sa_mlp(ns, g, rows, cent, wts):
    r, cp = rows.shape
    bs = cent.shape[0]
    h_out = wts[-2].shape[1]
    nw = len(wts) // 2
    w_specs = [pl.BlockSpec(w.shape, lambda i: tuple(0 for _ in w.shape)) for w in wts]
    return pl.pallas_call(
        functools.partial(_sa_mlp_kernel, ns, g, nw),
        grid=(bs // g,),
        in_specs=[
            pl.BlockSpec((g * ns, cp), lambda i: (i, 0)),
            pl.BlockSpec((g, cp), lambda i: (i, 0)),
        ] + w_specs,
        out_specs=pl.BlockSpec((g, h_out), lambda i: (i, 0)),
        out_shape=jax.ShapeDtypeStruct((bs, h_out), jnp.float32),
    )(rows, cent, *wts)


def _fp_kernel(nl, relu_flags, x1_ref, x2t_ref, f2_ref, f1_ref, *refs):
    w_refs = refs[:-1]
    out_ref = refs[-1]
    t = x1_ref.shape[1]
    n2 = x2t_ref.shape[2]
    sqr = jnp.zeros((t, n2), jnp.float32)
    for c in range(3):
        a = x1_ref[0, :, c:c + 1]
        b = x2t_ref[0, c:c + 1, :]
        sqr = sqr + (a - b) ** 2
    iota = lax.broadcasted_iota(jnp.int32, (t, n2), 1)
    wmat = jnp.zeros((t, n2), jnp.float32)
    wsum = jnp.zeros((t, 1), jnp.float32)
    for _ in range(3):
        m = jnp.min(sqr, axis=1, keepdims=True)
        j = jnp.min(jnp.where(sqr == m, iota, n2), axis=1, keepdims=True)
        occ = iota == j
        recip = 1.0 / (m + 1e-8)
        wmat = wmat + jnp.where(occ, recip, 0.0)
        wsum = wsum + recip
        sqr = jnp.where(occ, 1e30, sqr)
    wmat = wmat / wsum
    interp = jnp.dot(wmat, f2_ref[0], preferred_element_type=jnp.float32)
    x = jnp.concatenate([f1_ref[0], interp], axis=1)
    for li in range(nl):
        w = w_refs[2 * li][...]
        bb = w_refs[2 * li + 1][...]
        x = jnp.dot(x, w, preferred_element_type=jnp.float32) + bb
        if relu_flags[li]:
            x = jnp.maximum(x, 0.0)
    out_ref[0] = x


def _run_fp(x1_b, x2_t, f2, f1, wts, relu_flags):
    b, n1, _ = x1_b.shape
    n2 = x2_t.shape[2]
    c2 = f2.shape[2]
    c1 = f1.shape[2]
    h_out = wts[-2].shape[1]
    nl = len(wts) // 2
    t = min(n1, 1024)
    w_specs = [pl.BlockSpec(w.shape, lambda i, jj: tuple(0 for _ in w.shape)) for w in wts]
    return pl.pallas_call(
        functools.partial(_fp_kernel, nl, relu_flags),
        grid=(b, n1 // t),
        in_specs=[
            pl.BlockSpec((1, t, 3), lambda i, jj: (i, jj, 0)),
            pl.BlockSpec((1, 3, n2), lambda i, jj: (i, 0, 0)),
            pl.BlockSpec((1, n2, c2), lambda i, jj: (i, 0, 0)),
            pl.BlockSpec((1, t, c1), lambda i, jj: (i, jj, 0)),
        ] + w_specs,
        out_specs=pl.BlockSpec((1, t, h_out), lambda i, jj: (i, jj, 0)),
        out_shape=jax.ShapeDtypeStruct((b, n1, h_out), jnp.float32),
    )(x1_b, x2_t, f2, f1, *wts)


_SA_CFG = (
    (1024, 32, 0.1, 'sa1', 256, 8),
    (256, 32, 0.2, 'sa2', 256, 2),
    (64, 32, 0.4, 'sa3', 128, 2),
    (16, 32, 0.8, 'sa4', 128, 1),
)


def _prep_mlp(mlp, pad_in=None):
    wts = []
    for li, (w, bb) in enumerate(mlp):
        wt = w.T
        if li == 0 and pad_in is not None and pad_in > wt.shape[0]:
            wt = jnp.pad(wt, ((0, pad_in - wt.shape[0]), (0, 0)))
        wts.append(wt)
        wts.append(bb[None, :])
    return wts


def kernel(points, features, params):
    xyz_b = jnp.transpose(points, (0, 2, 1))
    feats_b = jnp.transpose(features, (0, 2, 1))

    nx = _run_fps(points)

    lvl_xyz_t = [points]
    lvl_xyz_b = [xyz_b]
    lvl_f = [feats_b]

    cur_xyz_t, cur_xyz_b, cur_f = points, xyz_b, feats_b
    for si, (s, ns, radius, key, g, nparts) in enumerate(_SA_CFG):
        n = cur_xyz_t.shape[2]
        nx_t = nx[si]
        nx_b = jnp.transpose(nx_t, (0, 2, 1))
        idx = _run_ballq(radius, ns, cur_xyz_t, nx_b)

        cin = 3 + cur_f.shape[2]
        cp = _ceil_to(cin, 128)
        table = jnp.concatenate([cur_xyz_b, cur_f], axis=2)
        table = jnp.pad(table, ((0, 0), (0, 0), (0, cp - cin))).reshape(_B * n, cp)
        flat = (idx + (jnp.arange(_B, dtype=jnp.int32) * n)[:, None, None]).reshape(-1)
        cent = jnp.pad(nx_b, ((0, 0), (0, 0), (0, cp - 3))).reshape(_B * s, cp)
        wts = _prep_mlp(params[key], pad_in=cp)

        gs = (_B * s) // nparts
        parts = []
        for p in range(nparts):
            rows_p = _gather_rows(table, lax.slice(flat, (p * gs * ns,), ((p + 1) * gs * ns,)))
            cent_p = lax.slice(cent, (p * gs, 0), ((p + 1) * gs, cp))
            parts.append(_run_sa_mlp(ns, g, rows_p, cent_p, wts))
        f_new = jnp.concatenate(parts, axis=0) if nparts > 1 else parts[0]

        cur_xyz_t, cur_xyz_b, cur_f = nx_t, nx_b, f_new.reshape(_B, s, -1)
        lvl_xyz_t.append(cur_xyz_t)
        lvl_xyz_b.append(cur_xyz_b)
        lvl_f.append(cur_f)

    fp_cfg = (
        (3, 4, 'fp1', None),
        (2, 3, 'fp2', None),
        (1, 2, 'fp3', None),
        (0, 1, 'fp4', 'cls'),
    )
    for dst, src, pkey, cls_key in fp_cfg:
        mlp = list(params[pkey])
        relu = [True] * len(mlp)
        if cls_key is not None:
            cw = list(params[cls_key])
            mlp = mlp + cw
            relu = relu + [True] * (len(cw) - 1) + [False]
        wts = _prep_mlp(mlp)
        out = _run_fp(lvl_xyz_b[dst], lvl_xyz_t[src], lvl_f[src], lvl_f[dst],
                      wts, tuple(relu))
        lvl_f[dst] = out

    return jnp.transpose(lvl_f[0], (0, 2, 1))


# speedup vs baseline: 1.0150x; 1.0150x over previous
import functools

import jax
import jax.numpy as jnp
from jax import lax
from jax.experimental import pallas as pl
from jax.experimental.pallas import tpu as pltpu
from jax.experimental.pallas import tpu_sc as plsc

_B = 8
_N0 = 4096


def _ceil_to(x, m):
    return (x + m - 1) // m * m


_FPS_SIZES = (1024, 256, 64, 16)


def _fps_kernel(pts_ref, o1, o2, o3, o4):
    def stage(src_ref, npoint, out_ref):
        x = src_ref[:, 0, :]
        y = src_ref[:, 1, :]
        z = src_ref[:, 2, :]
        n = x.shape[1]
        iota = lax.broadcasted_iota(jnp.int32, (_B, n), 1)
        iota3 = lax.broadcasted_iota(jnp.int32, (_B, 3, npoint), 2)

        def body(i, carry):
            dist, far, acc = carry
            oh = iota == far
            cx = jnp.sum(jnp.where(oh, x, 0.0), axis=1, keepdims=True)
            cy = jnp.sum(jnp.where(oh, y, 0.0), axis=1, keepdims=True)
            cz = jnp.sum(jnp.where(oh, z, 0.0), axis=1, keepdims=True)
            c3 = jnp.concatenate([cx, cy, cz], axis=1)[:, :, None]
            acc = jnp.where(iota3 == i, c3, acc)
            d = (x - cx) ** 2 + (y - cy) ** 2 + (z - cz) ** 2
            dist = jnp.minimum(dist, d)
            m = jnp.max(dist, axis=1, keepdims=True)
            far = jnp.min(jnp.where(dist == m, iota, n), axis=1, keepdims=True)
            return dist, far, acc

        _, _, acc = lax.fori_loop(
            0, npoint, body,
            (jnp.full((_B, n), 1e10, jnp.float32), jnp.zeros((_B, 1), jnp.int32),
             jnp.zeros((_B, 3, npoint), jnp.float32)),
        )
        out_ref[...] = acc

    stage(pts_ref, _FPS_SIZES[0], o1)
    stage(o1, _FPS_SIZES[1], o2)
    stage(o2, _FPS_SIZES[2], o3)
    stage(o3, _FPS_SIZES[3], o4)


def _run_fps(points):
    return pl.pallas_call(
        _fps_kernel,
        out_shape=[jax.ShapeDtypeStruct((_B, 3, s), jnp.float32) for s in _FPS_SIZES],
    )(points)


def _ballq_kernel(r2, ns, xyz_ref, nxyz_ref, idx_ref):
    s = nxyz_ref.shape[1]
    n = xyz_ref.shape[2]
    sqr = jnp.zeros((s, n), jnp.float32)
    for c in range(3):
        a = nxyz_ref[0, :, c:c + 1]
        b = xyz_ref[0, c:c + 1, :]
        sqr = sqr + (a - b) ** 2
    iota = lax.broadcasted_iota(jnp.int32, (s, n), 1)
    vals = jnp.where(sqr <= r2, iota, n)
    cols = []
    for _ in range(ns):
        m = jnp.min(vals, axis=1, keepdims=True)
        cols.append(m)
        vals = jnp.where(vals == m, n, vals)
    idx = jnp.concatenate(cols, axis=1)
    idx = jnp.where(idx == n, jnp.broadcast_to(idx[:, 0:1], idx.shape), idx)
    idx_ref[0] = idx


def _run_ballq(radius, ns, xyz_t, nxyz_b):
    b, _, n = xyz_t.shape
    s = nxyz_b.shape[1]
    return pl.pallas_call(
        functools.partial(_ballq_kernel, radius * radius, ns),
        grid=(b,),
        in_specs=[
            pl.BlockSpec((1, 3, n), lambda i: (i, 0, 0)),
            pl.BlockSpec((1, s, 3), lambda i: (i, 0, 0)),
        ],
        out_specs=pl.BlockSpec((1, s, ns), lambda i: (i, 0, 0)),
        out_shape=jax.ShapeDtypeStruct((b, s, ns), jnp.int32),
    )(xyz_t, nxyz_b)


def _gather_rows(table, flat_idx):
    r = flat_idx.shape[0]
    cp = table.shape[1]
    info = plsc.get_sparse_core_info()
    nw = info.num_cores * info.num_subcores
    nc = info.num_cores
    per_w = r // nw
    chunk = min(per_w, 128)
    steps = per_w // chunk
    mesh = plsc.VectorSubcoreMesh(core_axis_name="c", subcore_axis_name="s")

    @functools.partial(
        pl.kernel, mesh=mesh,
        out_type=jax.ShapeDtypeStruct((r, cp), jnp.float32),
        scratch_types=[
            pltpu.VMEM((chunk,), jnp.int32),
            pltpu.VMEM((chunk, cp), jnp.float32),
            pltpu.SemaphoreType.DMA,
        ],
    )
    def gk(table_hbm, idx_hbm, out_hbm, idx_v, rows_v, sem):
        wid = lax.axis_index("s") * nc + lax.axis_index("c")
        base = wid * per_w

        def body(t, carry):
            off = base + t * chunk
            pltpu.sync_copy(idx_hbm.at[pl.ds(off, chunk)], idx_v)
            pltpu.async_copy(table_hbm.at[idx_v], rows_v, sem).wait()
            pltpu.sync_copy(rows_v, out_hbm.at[pl.ds(off, chunk)])
            return carry

        lax.fori_loop(0, steps, body, 0)

    return gk(table, flat_idx)


def _sa_mlp_kernel(ns, g, nw, rows_ref, cent_ref, *refs):
    w_refs = refs[:-1]
    out_ref = refs[-1]
    cp = rows_ref.shape[1]
    x = rows_ref[...].reshape(g, ns, cp) - cent_ref[...][:, None, :]
    x = x.reshape(g * ns, cp)
    for li in range(nw):
        w = w_refs[2 * li][...]
        bb = w_refs[2 * li + 1][...]
        x = jnp.maximum(jnp.dot(x, w, preferred_element_type=jnp.float32) + bb, 0.0)
    h = x.shape[1]
    out_ref[...] = jnp.max(x.reshape(g, ns, h), axis=1)


def _run_sa_mlp(ns, g, rows, cent, wts):
    r, cp = rows.shape
    bs = cent.shape[0]
    h_out = wts[-2].shape[1]
    nw = len(wts) // 2
    w_specs = [pl.BlockSpec(w.shape, lambda i: tuple(0 for _ in w.shape)) for w in wts]
    return pl.pallas_call(
        functools.partial(_sa_mlp_kernel, ns, g, nw),
        grid=(bs // g,),
        in_specs=[
            pl.BlockSpec((g * ns, cp), lambda i: (i, 0)),
            pl.BlockSpec((g, cp), lambda i: (i, 0)),
        ] + w_specs,
        out_specs=pl.BlockSpec((g, h_out), lambda i: (i, 0)),
        out_shape=jax.ShapeDtypeStruct((bs, h_out), jnp.float32),
    )(rows, cent, *wts)


def _fp_kernel(nl, relu_flags, x1_ref, x2t_ref, f2_ref, f1_ref, *refs):
    w_refs = refs[:-1]
    out_ref = refs[-1]
    t = x1_ref.shape[1]
    n2 = x2t_ref.shape[2]
    sqr = jnp.zeros((t, n2), jnp.float32)
    for c in range(3):
        a = x1_ref[0, :, c:c + 1]
        b = x2t_ref[0, c:c + 1, :]
        sqr = sqr + (a - b) ** 2
    iota = lax.broadcasted_iota(jnp.int32, (t, n2), 1)
    wmat = jnp.zeros((t, n2), jnp.float32)
    wsum = jnp.zeros((t, 1), jnp.float32)
    for _ in range(3):
        m = jnp.min(sqr, axis=1, keepdims=True)
        j = jnp.min(jnp.where(sqr == m, iota, n2), axis=1, keepdims=True)
        occ = iota == j
        recip = 1.0 / (m + 1e-8)
        wmat = wmat + jnp.where(occ, recip, 0.0)
        wsum = wsum + recip
        sqr = jnp.where(occ, 1e30, sqr)
    wmat = wmat / wsum
    interp = jnp.dot(wmat, f2_ref[0], preferred_element_type=jnp.float32)
    x = jnp.concatenate([f1_ref[0], interp], axis=1)
    for li in range(nl):
        w = w_refs[2 * li][...]
        bb = w_refs[2 * li + 1][...]
        x = jnp.dot(x, w, preferred_element_type=jnp.float32) + bb
        if relu_flags[li]:
            x = jnp.maximum(x, 0.0)
    out_ref[0] = x


def _run_fp(x1_b, x2_t, f2, f1, wts, relu_flags):
    b, n1, _ = x1_b.shape
    n2 = x2_t.shape[2]
    c2 = f2.shape[2]
    c1 = f1.shape[2]
    h_out = wts[-2].shape[1]
    nl = len(wts) // 2
    t = min(n1, 1024)
    w_specs = [pl.BlockSpec(w.shape, lambda i, jj: tuple(0 for _ in w.shape)) for w in wts]
    return pl.pallas_call(
        functools.partial(_fp_kernel, nl, relu_flags),
        grid=(b, n1 // t),
        in_specs=[
            pl.BlockSpec((1, t, 3), lambda i, jj: (i, jj, 0)),
            pl.BlockSpec((1, 3, n2), lambda i, jj: (i, 0, 0)),
            pl.BlockSpec((1, n2, c2), lambda i, jj: (i, 0, 0)),
            pl.BlockSpec((1, t, c1), lambda i, jj: (i, jj, 0)),
        ] + w_specs,
        out_specs=pl.BlockSpec((1, t, h_out), lambda i, jj: (i, jj, 0)),
        out_shape=jax.ShapeDtypeStruct((b, n1, h_out), jnp.float32),
    )(x1_b, x2_t, f2, f1, *wts)


_SA_CFG = (
    (1024, 32, 0.1, 'sa1', 256, 4),
    (256, 32, 0.2, 'sa2', 256, 2),
    (64, 32, 0.4, 'sa3', 128, 2),
    (16, 32, 0.8, 'sa4', 128, 1),
)


def _prep_mlp(mlp, pad_in=None):
    wts = []
    for li, (w, bb) in enumerate(mlp):
        wt = w.T
        if li == 0 and pad_in is not None and pad_in > wt.shape[0]:
            wt = jnp.pad(wt, ((0, pad_in - wt.shape[0]), (0, 0)))
        wts.append(wt)
        wts.append(bb[None, :])
    return wts


def kernel(points, features, params):
    xyz_b = jnp.transpose(points, (0, 2, 1))
    feats_b = jnp.transpose(features, (0, 2, 1))

    nx = _run_fps(points)

    lvl_xyz_t = [points]
    lvl_xyz_b = [xyz_b]
    lvl_f = [feats_b]

    cur_xyz_t, cur_xyz_b, cur_f = points, xyz_b, feats_b
    for si, (s, ns, radius, key, g, nparts) in enumerate(_SA_CFG):
        n = cur_xyz_t.shape[2]
        nx_t = nx[si]
        nx_b = jnp.transpose(nx_t, (0, 2, 1))
        idx = _run_ballq(radius, ns, cur_xyz_t, nx_b)

        cin = 3 + cur_f.shape[2]
        cp = _ceil_to(cin, 128)
        table = jnp.concatenate([cur_xyz_b, cur_f], axis=2)
        table = jnp.pad(table, ((0, 0), (0, 0), (0, cp - cin))).reshape(_B * n, cp)
        flat = (idx + (jnp.arange(_B, dtype=jnp.int32) * n)[:, None, None]).reshape(-1)
        cent = jnp.pad(nx_b, ((0, 0), (0, 0), (0, cp - 3))).reshape(_B * s, cp)
        wts = _prep_mlp(params[key], pad_in=cp)

        gs = (_B * s) // nparts
        parts = []
        for p in range(nparts):
            rows_p = _gather_rows(table, lax.slice(flat, (p * gs * ns,), ((p + 1) * gs * ns,)))
            cent_p = lax.slice(cent, (p * gs, 0), ((p + 1) * gs, cp))
            parts.append(_run_sa_mlp(ns, g, rows_p, cent_p, wts))
        f_new = jnp.concatenate(parts, axis=0) if nparts > 1 else parts[0]

        cur_xyz_t, cur_xyz_b, cur_f = nx_t, nx_b, f_new.reshape(_B, s, -1)
        lvl_xyz_t.append(cur_xyz_t)
        lvl_xyz_b.append(cur_xyz_b)
        lvl_f.append(cur_f)

    fp_cfg = (
        (3, 4, 'fp1', None),
        (2, 3, 'fp2', None),
        (1, 2, 'fp3', None),
        (0, 1, 'fp4', 'cls'),
    )
    for dst, src, pkey, cls_key in fp_cfg:
        mlp = list(params[pkey])
        relu = [True] * len(mlp)
        if cls_key is not None:
            cw = list(params[cls_key])
            mlp = mlp + cw
            relu = relu + [True] * (len(cw) - 1) + [False]
        wts = _prep_mlp(mlp)
        out = _run_fp(lvl_xyz_b[dst], lvl_xyz_t[src], lvl_f[src], lvl_f[dst],
                      wts, tuple(relu))
        lvl_f[dst] = out

    return jnp.transpose(lvl_f[0], (0, 2, 1))
